# baseline (speedup 1.0000x reference)
import jax
import jax.numpy as jnp
from jax import lax
from jax.experimental import pallas as pl
from jax.experimental.pallas import tpu as pltpu
from jax.experimental.pallas import tpu_sc as plsc

B = 16384
D = 32
NUM_WORKERS = 32
BPW = B // NUM_WORKERS
CHUNK = 128
NCHUNK = BPW // CHUNK
GROUPS = BPW // 16


def _mf_body(uid_hbm, iid_hbm, utT_hbm, itT_hbm, out_hbm,
             uidx_v, iidx_v, ubuf_v, ibuf_v, res_v, sem):
    c = lax.axis_index("c")
    s = lax.axis_index("s")
    wid = s * 2 + c
    base = wid * BPW

    pltpu.sync_copy(uid_hbm.at[pl.ds(base, BPW)], uidx_v)
    pltpu.sync_copy(iid_hbm.at[pl.ds(base, BPW)], iidx_v)

    copies = []
    for col in range(D):
        for k in range(NCHUNK):
            sl = pl.ds(k * CHUNK, CHUNK)
            copies.append(pltpu.async_copy(
                utT_hbm.at[col].at[uidx_v.at[sl]], ubuf_v.at[col, sl], sem))
            copies.append(pltpu.async_copy(
                itT_hbm.at[col].at[iidx_v.at[sl]], ibuf_v.at[col, sl], sem))
    for cp in copies:
        cp.wait()

    def group(g, carry):
        r0 = g * 16
        acc = ubuf_v[0, pl.ds(r0, 16)] * ibuf_v[0, pl.ds(r0, 16)]
        for col in range(1, D):
            acc = acc + ubuf_v[col, pl.ds(r0, 16)] * ibuf_v[col, pl.ds(r0, 16)]
        res_v[pl.ds(r0, 16)] = 1.0 / (1.0 + jnp.exp(-acc))
        return carry

    lax.fori_loop(0, GROUPS, group, 0)

    pltpu.sync_copy(res_v, out_hbm.at[pl.ds(base, BPW)])


def kernel(user_ids, item_ids, user_table, item_table):
    mesh = plsc.VectorSubcoreMesh(core_axis_name="c", subcore_axis_name="s")
    mf = pl.kernel(
        _mf_body,
        mesh=mesh,
        out_type=jax.ShapeDtypeStruct((B,), jnp.float32),
        scratch_types=[
            pltpu.VMEM((BPW,), jnp.int32),
            pltpu.VMEM((BPW,), jnp.int32),
            pltpu.VMEM((D, BPW), jnp.float32),
            pltpu.VMEM((D, BPW), jnp.float32),
            pltpu.VMEM((BPW,), jnp.float32),
            pltpu.SemaphoreType.DMA,
        ],
        compiler_params=pltpu.CompilerParams(use_tc_tiling_on_sc=False),
    )
    return mf(user_ids.astype(jnp.int32), item_ids.astype(jnp.int32),
              user_table.T, item_table.T)


# speedup vs baseline: 3.6646x; 3.6646x over previous
import jax
import jax.numpy as jnp
from jax import lax
from jax.experimental import pallas as pl
from jax.experimental.pallas import tpu as pltpu
from jax.experimental.pallas import tpu_sc as plsc

B = 16384
D = 32
NUM_WORKERS = 32
BPW = B // NUM_WORKERS
CHUNK = 128
NCHUNK = BPW // CHUNK
GROUPS = BPW // 16


def _mf_body(*refs):
    uid_hbm, iid_hbm = refs[0], refs[1]
    ucols = refs[2:2 + D]
    icols = refs[2 + D:2 + 2 * D]
    out_hbm = refs[2 + 2 * D]
    uidx_v, iidx_v, ubuf_v, ibuf_v, res_v, sem = refs[3 + 2 * D:]

    c = lax.axis_index("c")
    s = lax.axis_index("s")
    wid = s * 2 + c
    base = wid * BPW

    pltpu.sync_copy(uid_hbm.at[pl.ds(base, BPW)], uidx_v)
    pltpu.sync_copy(iid_hbm.at[pl.ds(base, BPW)], iidx_v)

    copies = []
    for col in range(D):
        for k in range(NCHUNK):
            sl = pl.ds(k * CHUNK, CHUNK)
            copies.append(pltpu.async_copy(
                ucols[col].at[uidx_v.at[sl]], ubuf_v.at[col, sl], sem))
            copies.append(pltpu.async_copy(
                icols[col].at[iidx_v.at[sl]], ibuf_v.at[col, sl], sem))
    for cp in copies:
        cp.wait()

    def group(g, carry):
        r0 = g * 16
        acc = ubuf_v[0, pl.ds(r0, 16)] * ibuf_v[0, pl.ds(r0, 16)]
        for col in range(1, D):
            acc = acc + ubuf_v[col, pl.ds(r0, 16)] * ibuf_v[col, pl.ds(r0, 16)]
        res_v[pl.ds(r0, 16)] = 1.0 / (1.0 + jnp.exp(-acc))
        return carry

    lax.fori_loop(0, GROUPS, group, 0)

    pltpu.sync_copy(res_v, out_hbm.at[pl.ds(base, BPW)])


def kernel(user_ids, item_ids, user_table, item_table):
    mesh = plsc.VectorSubcoreMesh(core_axis_name="c", subcore_axis_name="s")
    mf = pl.kernel(
        _mf_body,
        mesh=mesh,
        out_type=jax.ShapeDtypeStruct((B,), jnp.float32),
        scratch_types=[
            pltpu.VMEM((BPW,), jnp.int32),
            pltpu.VMEM((BPW,), jnp.int32),
            pltpu.VMEM((D, BPW), jnp.float32),
            pltpu.VMEM((D, BPW), jnp.float32),
            pltpu.VMEM((BPW,), jnp.float32),
            pltpu.SemaphoreType.DMA,
        ],
        compiler_params=pltpu.CompilerParams(use_tc_tiling_on_sc=False),
    )
    ucols = [user_table[:, col] for col in range(D)]
    icols = [item_table[:, col] for col in range(D)]
    return mf(user_ids.astype(jnp.int32), item_ids.astype(jnp.int32),
              *ucols, *icols)


# speedup vs baseline: 5.7369x; 1.5655x over previous
import jax
import jax.numpy as jnp
from jax import lax
from jax.experimental import pallas as pl
from jax.experimental.pallas import tpu as pltpu
from jax.experimental.pallas import tpu_sc as plsc

B = 16384
D = 32
NUM_WORKERS = 32
BPW = B // NUM_WORKERS
CHUNK = 128
NCHUNK = BPW // CHUNK
GROUPS = BPW // 16

def _perm(x, p):
    return x.at[p].get(mode="promise_in_bounds")


def _merge(a, b, m, lanes):
    h = m // 2
    p = lanes ^ h
    pa = a + _perm(a, p)
    pb = b + _perm(b, p)
    mask = (lanes & h) == 0
    return jnp.where(mask, pa, _perm(pb, p))


def _rowsums16(vecs, lanes):
    m = 16
    while len(vecs) > 1:
        half = len(vecs) // 2
        vecs = [_merge(vecs[i], vecs[i + half], m, lanes) for i in range(half)]
        m //= 2
    return vecs[0]


def _mf_body(uid_hbm, iid_hbm, ut_hbm, it_hbm, out_hbm,
             uidx_v, iidx_v, urows_v, irows_v, res_v, sem):
    c = lax.axis_index("c")
    s = lax.axis_index("s")
    wid = s * 2 + c
    base = wid * BPW

    pltpu.sync_copy(uid_hbm.at[pl.ds(base, BPW)], uidx_v)
    pltpu.sync_copy(iid_hbm.at[pl.ds(base, BPW)], iidx_v)

    copies = []
    for k in range(NCHUNK):
        sl = pl.ds(k * CHUNK, CHUNK)
        copies.append(pltpu.async_copy(ut_hbm.at[uidx_v.at[sl]], urows_v.at[sl], sem))
        copies.append(pltpu.async_copy(it_hbm.at[iidx_v.at[sl]], irows_v.at[sl], sem))
    for cp in copies:
        cp.wait()

    lanes = lax.iota(jnp.int32, 16)

    def group(g, carry):
        row0 = g * 16
        prods = []
        for r in range(16):
            row = row0 + r
            t = (urows_v[row, pl.ds(0, 16)] * irows_v[row, pl.ds(0, 16)]
                 + urows_v[row, pl.ds(16, 16)] * irows_v[row, pl.ds(16, 16)])
            prods.append(t)
        dots = _rowsums16(prods, lanes)
        res_v[pl.ds(row0, 16)] = 1.0 / (1.0 + jnp.exp(-dots))
        return carry

    lax.fori_loop(0, GROUPS, group, 0)

    pltpu.sync_copy(res_v, out_hbm.at[pl.ds(base, BPW)])


def kernel(user_ids, item_ids, user_table, item_table):
    mesh = plsc.VectorSubcoreMesh(core_axis_name="c", subcore_axis_name="s")
    mf = pl.kernel(
        _mf_body,
        mesh=mesh,
        out_type=jax.ShapeDtypeStruct((B,), jnp.float32),
        scratch_types=[
            pltpu.VMEM((BPW,), jnp.int32),
            pltpu.VMEM((BPW,), jnp.int32),
            pltpu.VMEM((BPW, D), jnp.float32),
            pltpu.VMEM((BPW, D), jnp.float32),
            pltpu.VMEM((BPW,), jnp.float32),
            pltpu.SemaphoreType.DMA,
        ],
        compiler_params=pltpu.CompilerParams(use_tc_tiling_on_sc=False),
    )
    return mf(user_ids.astype(jnp.int32), item_ids.astype(jnp.int32),
              user_table, item_table)
